# baseline (speedup 1.0000x reference)
import jax
import jax.numpy as jnp
from jax import lax
from jax.experimental import pallas as pl
from jax.experimental.pallas import tpu as pltpu
from jax.experimental.pallas import tpu_sc as plsc

B = 16384
D = 64
NC = 2
NS = 16
NW = NC * NS
BPW = B // NW
CHUNK = 128
NCHUNK = BPW // CHUNK
NV = 100000


def _concat_body(a_ref, b_ref, out_ref):
    out_ref[...] = jnp.concatenate([a_ref[...], b_ref[...]], axis=1)


RBS = 2000


def _tc_concat(a, b):
    grid = (NV // RBS,)
    spec = pl.BlockSpec((RBS, D), lambda i: (i, 0))
    return pl.pallas_call(
        _concat_body,
        grid=grid,
        in_specs=[spec, spec],
        out_specs=pl.BlockSpec((RBS, 2 * D), lambda i: (i, 0)),
        out_shape=jax.ShapeDtypeStruct((NV, 2 * D), jnp.float32),
    )(a, b)


def _sc_gather_body(ids_hbm, tbl_hbm, out, idx, rows, sem0, sem1, wsem):
    wid = lax.axis_index("s") * NC + lax.axis_index("c")
    base = wid * BPW

    for j in range(NCHUNK):
        pltpu.sync_copy(ids_hbm.at[pl.ds(base + j * CHUNK, CHUNK)], idx.at[j])

    sems = (sem0, sem1)
    n = NCHUNK
    descs = [None] * n
    wd = [None] * n

    descs[0] = pltpu.async_copy(tbl_hbm.at[idx.at[0]], rows.at[0], sems[0])
    for k in range(n):
        buf = k % 2
        if k + 1 < n:
            if k - 1 >= 0:
                wd[k - 1].wait()
            descs[k + 1] = pltpu.async_copy(
                tbl_hbm.at[idx.at[k + 1]], rows.at[1 - buf], sems[1 - buf])
        descs[k].wait()
        wd[k] = pltpu.async_copy(
            rows.at[buf], out.at[pl.ds(base + k * CHUNK, CHUNK)], wsem)
    wd[n - 2].wait()
    wd[n - 1].wait()


@jax.jit
def _sc_gather(ids, tbl):
    mesh = plsc.VectorSubcoreMesh(core_axis_name="c", subcore_axis_name="s")
    f = pl.kernel(
        _sc_gather_body,
        out_type=jax.ShapeDtypeStruct((B, 2 * D), jnp.float32),
        mesh=mesh,
        scratch_types=[
            pltpu.VMEM((NCHUNK, CHUNK), jnp.int32),
            pltpu.VMEM((2, CHUNK, 2 * D), jnp.float32),
            pltpu.SemaphoreType.DMA,
            pltpu.SemaphoreType.DMA,
            pltpu.SemaphoreType.DMA,
        ],
        compiler_params=pltpu.CompilerParams(use_tc_tiling_on_sc=True),
    )
    return f(ids, tbl)


def _mlp_body(xu_ref, xi_ref, w0_ref, b0_ref, w1_ref, b1_ref,
              w2_ref, b2_ref, wp_ref, out_ref):
    xu = xu_ref[...]
    xi = xi_ref[...]
    h = jnp.concatenate([xu[:, :D], xi[:, :D]], axis=1)
    h = jnp.maximum(
        jnp.dot(h, w0_ref[...], preferred_element_type=jnp.float32)
        + b0_ref[...], 0.0)
    h = jnp.maximum(
        jnp.dot(h, w1_ref[...], preferred_element_type=jnp.float32)
        + b1_ref[...], 0.0)
    h = jnp.maximum(
        jnp.dot(h, w2_ref[...], preferred_element_type=jnp.float32)
        + b2_ref[...], 0.0)
    gmf = xu[:, D:] * xi[:, D:]
    out_ref[...] = (
        jnp.dot(gmf, wp_ref[0:D, :], preferred_element_type=jnp.float32)
        + jnp.dot(h, wp_ref[D:2 * D, :], preferred_element_type=jnp.float32))


BM = 2048


def _tc_mlp(xu, xi, W0, b0, W1, b1, W2, b2, Wp, interpret=False):
    grid = (B // BM,)
    row_spec = pl.BlockSpec((BM, 2 * D), lambda i: (i, 0))
    full = lambda shape: pl.BlockSpec(shape, lambda i: tuple(0 for _ in shape))
    return pl.pallas_call(
        _mlp_body,
        grid=grid,
        in_specs=[
            row_spec, row_spec,
            full(W0.shape), full((1, 256)),
            full(W1.shape), full((1, 128)),
            full(W2.shape), full((1, 64)),
            full((128, 1)),
        ],
        out_specs=pl.BlockSpec((BM, 1), lambda i: (i, 0)),
        out_shape=jax.ShapeDtypeStruct((B, 1), jnp.float32),
        interpret=interpret,
    )(xu, xi, W0, b0.reshape(1, -1), W1, b1.reshape(1, -1),
      W2, b2.reshape(1, -1), Wp)


def kernel(user_id, item_id, u_mlp, i_mlp, u_gmf, i_gmf,
           W0, b0, W1, b1, W2, b2, Wp):
    cu = _tc_concat(u_mlp, u_gmf)
    xu = _sc_gather(user_id, cu)
    ci = _tc_concat(i_mlp, i_gmf)
    xi = _sc_gather(item_id, ci)
    return _tc_mlp(xu, xi, W0, b0, W1, b1, W2, b2, Wp).reshape(-1)
